# Initial kernel scaffold; baseline (speedup 1.0000x reference)
import jax
import jax.numpy as jnp
from jax.experimental import pallas as pl


def kernel(x, weight):
    raise NotImplementedError("write your pallas kernel here")


# speedup vs baseline: 2.7989x; 2.7989x over previous
import jax
import jax.numpy as jnp
from jax.experimental import pallas as pl
from jax.experimental.pallas import tpu as pltpu

_H = 512
_W = 512


def _dw3x3_kernel(w_ref, x_ref, o_ref):
    c = pl.program_id(1)
    x = x_ref[0, 0]
    zrow = jnp.zeros((1, _W), jnp.float32)
    zcol = jnp.zeros((_H, 1), jnp.float32)
    acc = None
    for ky in range(3):
        if ky == 0:
            xs = jnp.concatenate([zrow, x[:-1, :]], axis=0)
        elif ky == 1:
            xs = x
        else:
            xs = jnp.concatenate([x[1:, :], zrow], axis=0)
        for kx in range(3):
            if kx == 0:
                xss = jnp.concatenate([zcol, xs[:, :-1]], axis=1)
            elif kx == 1:
                xss = xs
            else:
                xss = jnp.concatenate([xs[:, 1:], zcol], axis=1)
            w = w_ref[c, ky * 3 + kx]
            term = xss * w
            acc = term if acc is None else acc + term
    o_ref[0, 0] = acc


def kernel(x, weight):
    n, ch, h, w = x.shape
    wmat = weight.reshape(ch, 9)
    grid = (n, ch)
    return pl.pallas_call(
        _dw3x3_kernel,
        grid=grid,
        in_specs=[
            pl.BlockSpec(memory_space=pltpu.SMEM),
            pl.BlockSpec((1, 1, h, w), lambda i, j: (i, j, 0, 0)),
        ],
        out_specs=pl.BlockSpec((1, 1, h, w), lambda i, j: (i, j, 0, 0)),
        out_shape=jax.ShapeDtypeStruct((n, ch, h, w), x.dtype),
    )(wmat, x)


# speedup vs baseline: 3.9870x; 1.4245x over previous
import jax
import jax.numpy as jnp
from jax.experimental import pallas as pl
from jax.experimental.pallas import tpu as pltpu

_H = 512
_W = 512


def _dw3x3_kernel(w_ref, x_ref, o_ref):
    c = pl.program_id(1)
    x = x_ref[0, 0]
    zrow = jnp.zeros((1, _W), jnp.float32)
    zcol = jnp.zeros((_H, 1), jnp.float32)
    xl = jnp.concatenate([zcol, x[:, :-1]], axis=1)
    xr = jnp.concatenate([x[:, 1:], zcol], axis=1)
    w = [w_ref[c, k] for k in range(9)]
    h0 = w[0] * xl + w[1] * x + w[2] * xr
    h1 = w[3] * xl + w[4] * x + w[5] * xr
    h2 = w[6] * xl + w[7] * x + w[8] * xr
    o_ref[0, 0] = (
        h1
        + jnp.concatenate([zrow, h0[:-1, :]], axis=0)
        + jnp.concatenate([h2[1:, :], zrow], axis=0)
    )


def kernel(x, weight):
    n, ch, h, w = x.shape
    wmat = weight.reshape(ch, 9)
    grid = (n, ch)
    return pl.pallas_call(
        _dw3x3_kernel,
        grid=grid,
        in_specs=[
            pl.BlockSpec(memory_space=pltpu.SMEM),
            pl.BlockSpec((1, 1, h, w), lambda i, j: (i, j, 0, 0)),
        ],
        out_specs=pl.BlockSpec((1, 1, h, w), lambda i, j: (i, j, 0, 0)),
        out_shape=jax.ShapeDtypeStruct((n, ch, h, w), x.dtype),
    )(wmat, x)


# speedup vs baseline: 5.6304x; 1.4122x over previous
import jax
import jax.numpy as jnp
from jax.experimental import pallas as pl
from jax.experimental.pallas import tpu as pltpu

_H = 512
_W = 512


_CB = 4


def _dw3x3_kernel(w_ref, x_ref, o_ref):
    zrow = jnp.zeros((1, _W), jnp.float32)
    zcol = jnp.zeros((_H, 1), jnp.float32)
    for ch in range(_CB):
        c = pl.program_id(1) * _CB + ch
        x = x_ref[0, ch]
        xl = jnp.concatenate([zcol, x[:, :-1]], axis=1)
        xr = jnp.concatenate([x[:, 1:], zcol], axis=1)
        w = [w_ref[c, k] for k in range(9)]
        h0 = w[0] * xl + w[1] * x + w[2] * xr
        h1 = w[3] * xl + w[4] * x + w[5] * xr
        h2 = w[6] * xl + w[7] * x + w[8] * xr
        o_ref[0, ch] = (
            h1
            + jnp.concatenate([zrow, h0[:-1, :]], axis=0)
            + jnp.concatenate([h2[1:, :], zrow], axis=0)
        )


def kernel(x, weight):
    n, ch, h, w = x.shape
    wmat = weight.reshape(ch, 9)
    grid = (n, ch // _CB)
    return pl.pallas_call(
        _dw3x3_kernel,
        grid=grid,
        in_specs=[
            pl.BlockSpec(memory_space=pltpu.SMEM),
            pl.BlockSpec((1, _CB, h, w), lambda i, j: (i, j, 0, 0)),
        ],
        out_specs=pl.BlockSpec((1, _CB, h, w), lambda i, j: (i, j, 0, 0)),
        out_shape=jax.ShapeDtypeStruct((n, ch, h, w), x.dtype),
    )(wmat, x)


# speedup vs baseline: 5.7271x; 1.0172x over previous
import jax
import jax.numpy as jnp
from jax.experimental import pallas as pl
from jax.experimental.pallas import tpu as pltpu

_H = 512
_W = 512


_CB = 8


def _dw3x3_kernel(w_ref, x_ref, o_ref):
    zrow = jnp.zeros((1, _W), jnp.float32)
    zcol = jnp.zeros((_H, 1), jnp.float32)
    for ch in range(_CB):
        c = pl.program_id(1) * _CB + ch
        x = x_ref[0, ch]
        xl = jnp.concatenate([zcol, x[:, :-1]], axis=1)
        xr = jnp.concatenate([x[:, 1:], zcol], axis=1)
        w = [w_ref[c, k] for k in range(9)]
        h0 = w[0] * xl + w[1] * x + w[2] * xr
        h1 = w[3] * xl + w[4] * x + w[5] * xr
        h2 = w[6] * xl + w[7] * x + w[8] * xr
        o_ref[0, ch] = (
            h1
            + jnp.concatenate([zrow, h0[:-1, :]], axis=0)
            + jnp.concatenate([h2[1:, :], zrow], axis=0)
        )


def kernel(x, weight):
    n, ch, h, w = x.shape
    wmat = weight.reshape(ch, 9)
    grid = (n, ch // _CB)
    return pl.pallas_call(
        _dw3x3_kernel,
        grid=grid,
        in_specs=[
            pl.BlockSpec(memory_space=pltpu.SMEM),
            pl.BlockSpec((1, _CB, h, w), lambda i, j: (i, j, 0, 0)),
        ],
        out_specs=pl.BlockSpec((1, _CB, h, w), lambda i, j: (i, j, 0, 0)),
        out_shape=jax.ShapeDtypeStruct((n, ch, h, w), x.dtype),
    )(wmat, x)
